# baseline (speedup 1.0000x reference)
import functools

import jax
import jax.numpy as jnp
from jax.experimental import pallas as pl
from jax.experimental.pallas import tpu as pltpu

N_TOKENS = 4096
IN_DIM = 1024
HID_DIM = 512
OUT_DIM = 1024
N_EXPERTS = 8
LANES = 128

NEG = -1e30


def _router_body(x_ref, wr_ref, br_ref, w_ref):
    logits = jnp.dot(x_ref[...], wr_ref[...],
                     preferred_element_type=jnp.float32) + br_ref[...]
    lane = jax.lax.broadcasted_iota(jnp.int32, logits.shape, 1)
    logits = jnp.where(lane < N_EXPERTS, logits, NEG)
    m1 = jnp.max(logits, axis=1, keepdims=True)
    i1 = jnp.min(jnp.where(logits == m1, lane, LANES), axis=1, keepdims=True)
    l2 = jnp.where(lane == i1, NEG, logits)
    m2 = jnp.max(l2, axis=1, keepdims=True)
    i2 = jnp.min(jnp.where(l2 == m2, lane, LANES), axis=1, keepdims=True)
    e2 = jnp.exp(m2 - m1)
    w0 = 1.0 / (1.0 + e2)
    w1 = e2 * w0
    w_ref[...] = (jnp.where(lane == i1, w0, 0.0)
                  + jnp.where(lane == i2, w1, 0.0))


def _moe_body(x_ref, w1_ref, b1_ref, w2_ref, b2_ref, wts_ref, out_ref):
    e = pl.program_id(1)
    h = jnp.dot(x_ref[...], w1_ref[0],
                preferred_element_type=jnp.float32) + b1_ref[0]
    h = jnp.maximum(h, 0.0)
    y = jnp.dot(h.astype(jnp.bfloat16), w2_ref[0],
                preferred_element_type=jnp.float32) + b2_ref[0]
    lane = jax.lax.broadcasted_iota(jnp.int32, wts_ref.shape, 1)
    w_col = jnp.sum(jnp.where(lane == e, wts_ref[...], 0.0), axis=1,
                    keepdims=True)
    acc = y * w_col

    @pl.when(e == 0)
    def _init():
        out_ref[...] = acc

    @pl.when(e != 0)
    def _acc():
        out_ref[...] += acc


@jax.jit
def kernel(x, Wr, br, W1, b1, W2, b2):
    wr_pad = jnp.zeros((IN_DIM, LANES), jnp.float32).at[:, :N_EXPERTS].set(Wr)
    br_pad = jnp.zeros((1, LANES), jnp.float32).at[0, :N_EXPERTS].set(br)

    rt = 512
    wts = pl.pallas_call(
        _router_body,
        grid=(N_TOKENS // rt,),
        in_specs=[
            pl.BlockSpec((rt, IN_DIM), lambda t: (t, 0)),
            pl.BlockSpec((IN_DIM, LANES), lambda t: (0, 0)),
            pl.BlockSpec((1, LANES), lambda t: (0, 0)),
        ],
        out_specs=pl.BlockSpec((rt, LANES), lambda t: (t, 0)),
        out_shape=jax.ShapeDtypeStruct((N_TOKENS, LANES), jnp.float32),
    )(x, wr_pad, br_pad)

    mt = 1024
    out = pl.pallas_call(
        _moe_body,
        grid=(N_TOKENS // mt, N_EXPERTS),
        in_specs=[
            pl.BlockSpec((mt, IN_DIM), lambda t, e: (t, 0)),
            pl.BlockSpec((1, IN_DIM, HID_DIM), lambda t, e: (e, 0, 0)),
            pl.BlockSpec((1, 1, HID_DIM), lambda t, e: (e, 0, 0)),
            pl.BlockSpec((1, HID_DIM, OUT_DIM), lambda t, e: (e, 0, 0)),
            pl.BlockSpec((1, 1, OUT_DIM), lambda t, e: (e, 0, 0)),
            pl.BlockSpec((mt, LANES), lambda t, e: (t, 0)),
        ],
        out_specs=pl.BlockSpec((mt, OUT_DIM), lambda t, e: (t, 0)),
        out_shape=jax.ShapeDtypeStruct((N_TOKENS, OUT_DIM), jnp.float32),
    )(x.astype(jnp.bfloat16), W1.astype(jnp.bfloat16), b1[:, None, :],
      W2.astype(jnp.bfloat16), b2[:, None, :], wts)

    return out, wts[:, :N_EXPERTS]


# speedup vs baseline: 1.1893x; 1.1893x over previous
import functools

import jax
import jax.numpy as jnp
from jax.experimental import pallas as pl
from jax.experimental.pallas import tpu as pltpu

N_TOKENS = 4096
IN_DIM = 1024
HID_DIM = 512
OUT_DIM = 1024
N_EXPERTS = 8
LANES = 128

NEG = -1e30


def _router_body(x_ref, wr_ref, br_ref, w_ref):
    logits = jnp.dot(x_ref[...], wr_ref[...],
                     preferred_element_type=jnp.float32) + br_ref[...]
    lane = jax.lax.broadcasted_iota(jnp.int32, logits.shape, 1)
    logits = jnp.where(lane < N_EXPERTS, logits, NEG)
    m1 = jnp.max(logits, axis=1, keepdims=True)
    i1 = jnp.min(jnp.where(logits == m1, lane, LANES), axis=1, keepdims=True)
    l2 = jnp.where(lane == i1, NEG, logits)
    m2 = jnp.max(l2, axis=1, keepdims=True)
    i2 = jnp.min(jnp.where(l2 == m2, lane, LANES), axis=1, keepdims=True)
    e2 = jnp.exp(m2 - m1)
    w0 = 1.0 / (1.0 + e2)
    w1 = e2 * w0
    w_ref[...] = (jnp.where(lane == i1, w0, 0.0)
                  + jnp.where(lane == i2, w1, 0.0))


def _moe_body(x_ref, w1_ref, b1_ref, w2_ref, b2_ref, wts_ref, out_ref):
    e = pl.program_id(1)
    xb = x_ref[...].astype(jnp.bfloat16)
    h = jnp.dot(xb, w1_ref[0].astype(jnp.bfloat16),
                preferred_element_type=jnp.float32) + b1_ref[0]
    h = jnp.maximum(h, 0.0)
    y = jnp.dot(h.astype(jnp.bfloat16), w2_ref[0].astype(jnp.bfloat16),
                preferred_element_type=jnp.float32) + b2_ref[0]
    lane = jax.lax.broadcasted_iota(jnp.int32, wts_ref.shape, 1)
    w_col = jnp.sum(jnp.where(lane == e, wts_ref[...], 0.0), axis=1,
                    keepdims=True)
    acc = y * w_col

    @pl.when(e == 0)
    def _init():
        out_ref[...] = acc

    @pl.when(e != 0)
    def _acc():
        out_ref[...] += acc


@jax.jit
def kernel(x, Wr, br, W1, b1, W2, b2):
    wr_pad = jnp.zeros((IN_DIM, LANES), jnp.float32).at[:, :N_EXPERTS].set(Wr)
    br_pad = jnp.zeros((1, LANES), jnp.float32).at[0, :N_EXPERTS].set(br)

    rt = 512
    wts = pl.pallas_call(
        _router_body,
        grid=(N_TOKENS // rt,),
        in_specs=[
            pl.BlockSpec((rt, IN_DIM), lambda t: (t, 0)),
            pl.BlockSpec((IN_DIM, LANES), lambda t: (0, 0)),
            pl.BlockSpec((1, LANES), lambda t: (0, 0)),
        ],
        out_specs=pl.BlockSpec((rt, LANES), lambda t: (t, 0)),
        out_shape=jax.ShapeDtypeStruct((N_TOKENS, LANES), jnp.float32),
    )(x, wr_pad, br_pad)

    mt = 1024
    out = pl.pallas_call(
        _moe_body,
        grid=(N_TOKENS // mt, N_EXPERTS),
        in_specs=[
            pl.BlockSpec((mt, IN_DIM), lambda t, e: (t, 0)),
            pl.BlockSpec((1, IN_DIM, HID_DIM), lambda t, e: (e, 0, 0)),
            pl.BlockSpec((1, 1, HID_DIM), lambda t, e: (e, 0, 0)),
            pl.BlockSpec((1, HID_DIM, OUT_DIM), lambda t, e: (e, 0, 0)),
            pl.BlockSpec((1, 1, OUT_DIM), lambda t, e: (e, 0, 0)),
            pl.BlockSpec((mt, LANES), lambda t, e: (t, 0)),
        ],
        out_specs=pl.BlockSpec((mt, OUT_DIM), lambda t, e: (t, 0)),
        out_shape=jax.ShapeDtypeStruct((N_TOKENS, OUT_DIM), jnp.float32),
    )(x, W1, b1[:, None, :], W2, b2[:, None, :], wts)

    return out, wts[:, :N_EXPERTS]


# speedup vs baseline: 1.4078x; 1.1837x over previous
import jax
import jax.numpy as jnp
from jax.experimental import pallas as pl

N_TOKENS = 4096
IN_DIM = 1024
HID_DIM = 512
OUT_DIM = 1024
N_EXPERTS = 8
LANES = 128

NEG = -1e30


def _moe_body(x_ref, wr_ref, br_ref, w1_ref, b1_ref, w2_ref, b2_ref,
              out_ref, wts_ref):
    xf = x_ref[...]
    logits = jnp.dot(xf, wr_ref[...],
                     preferred_element_type=jnp.float32) + br_ref[...]
    lane = jax.lax.broadcasted_iota(jnp.int32, logits.shape, 1)
    logits = jnp.where(lane < N_EXPERTS, logits, NEG)
    m1 = jnp.max(logits, axis=1, keepdims=True)
    i1 = jnp.min(jnp.where(logits == m1, lane, LANES), axis=1, keepdims=True)
    l2 = jnp.where(lane == i1, NEG, logits)
    m2 = jnp.max(l2, axis=1, keepdims=True)
    i2 = jnp.min(jnp.where(l2 == m2, lane, LANES), axis=1, keepdims=True)
    e2 = jnp.exp(m2 - m1)
    w0 = 1.0 / (1.0 + e2)
    w1w = e2 * w0
    wts = (jnp.where(lane == i1, w0, 0.0) + jnp.where(lane == i2, w1w, 0.0))
    wts_ref[...] = wts

    xb = xf.astype(jnp.bfloat16)
    acc = None
    for e in range(N_EXPERTS):
        h = jnp.dot(xb, w1_ref[e].astype(jnp.bfloat16),
                    preferred_element_type=jnp.float32) + b1_ref[e]
        h = jnp.maximum(h, 0.0)
        y = jnp.dot(h.astype(jnp.bfloat16), w2_ref[e].astype(jnp.bfloat16),
                    preferred_element_type=jnp.float32) + b2_ref[e]
        w_col = jnp.sum(jnp.where(lane == e, wts, 0.0), axis=1, keepdims=True)
        acc = y * w_col if acc is None else acc + y * w_col
    out_ref[...] = acc


@jax.jit
def kernel(x, Wr, br, W1, b1, W2, b2):
    wr_pad = jnp.zeros((IN_DIM, LANES), jnp.float32).at[:, :N_EXPERTS].set(Wr)
    br_pad = jnp.zeros((1, LANES), jnp.float32).at[0, :N_EXPERTS].set(br)

    mt = 512
    out, wts = pl.pallas_call(
        _moe_body,
        grid=(N_TOKENS // mt,),
        in_specs=[
            pl.BlockSpec((mt, IN_DIM), lambda t: (t, 0)),
            pl.BlockSpec((IN_DIM, LANES), lambda t: (0, 0)),
            pl.BlockSpec((1, LANES), lambda t: (0, 0)),
            pl.BlockSpec((N_EXPERTS, IN_DIM, HID_DIM), lambda t: (0, 0, 0)),
            pl.BlockSpec((N_EXPERTS, 1, HID_DIM), lambda t: (0, 0, 0)),
            pl.BlockSpec((N_EXPERTS, HID_DIM, OUT_DIM), lambda t: (0, 0, 0)),
            pl.BlockSpec((N_EXPERTS, 1, OUT_DIM), lambda t: (0, 0, 0)),
        ],
        out_specs=[
            pl.BlockSpec((mt, OUT_DIM), lambda t: (t, 0)),
            pl.BlockSpec((mt, LANES), lambda t: (t, 0)),
        ],
        out_shape=[
            jax.ShapeDtypeStruct((N_TOKENS, OUT_DIM), jnp.float32),
            jax.ShapeDtypeStruct((N_TOKENS, LANES), jnp.float32),
        ],
    )(x, wr_pad, br_pad, W1, b1[:, None, :], W2, b2[:, None, :])

    return out, wts[:, :N_EXPERTS]
